# baseline (speedup 1.0000x reference)
import functools

import jax
import jax.numpy as jnp
from jax import lax
from jax.experimental import pallas as pl
from jax.experimental.pallas import tpu as pltpu
from jax.experimental.pallas import tpu_sc as plsc

NC = 2
NS = 16
NW = NC * NS
L = 16

BN = 2000
KC = 125


@functools.lru_cache(maxsize=None)
def _deg_kernel(NCH, NPAD):
    K = KC
    RPT = NPAD // NS

    mesh = plsc.VectorSubcoreMesh(core_axis_name="c", subcore_axis_name="s")

    @functools.partial(
        pl.kernel,
        out_type=jax.ShapeDtypeStruct((NC, NPAD), jnp.float32),
        mesh=mesh,
        scratch_types=[
            pltpu.VMEM((NCH, K), jnp.int32),
            pltpu.VMEM((128,), jnp.float32),
            pltpu.VMEM((RPT,), jnp.float32),
            pltpu.VMEM_SHARED((NPAD,), jnp.float32),
        ],
    )
    def deg_k(dst_hbm, out_hbm, dst_v, ones_v, z_v, acc_sh):
        c = lax.axis_index("c")
        s = lax.axis_index("s")
        wid = c * NS + s
        one16 = jnp.full((L,), 1.0, jnp.float32)
        zero16 = jnp.zeros((L,), jnp.float32)
        for k in range(128 // L):
            ones_v[pl.ds(k * L, L)] = one16
        for k in range(RPT // L):
            z_v[pl.ds(k * L, L)] = zero16
        pltpu.sync_copy(z_v, acc_sh.at[pl.ds(s * RPT, RPT)])
        pltpu.sync_copy(dst_hbm.at[wid], dst_v)
        plsc.subcore_barrier()

        @pl.loop(0, NCH)
        def _(j):
            pltpu.sync_copy(ones_v.at[pl.ds(0, K)], acc_sh.at[dst_v.at[j]],
                            add=True)

        plsc.subcore_barrier()
        pltpu.sync_copy(acc_sh.at[pl.ds(s * RPT, RPT)],
                        out_hbm.at[c, pl.ds(s * RPT, RPT)])

    return deg_k


@functools.lru_cache(maxsize=None)
def _scatter_kernel(NCH, NPAD, D):
    K = KC
    RPT = NPAD // NS
    ZR = 32
    NST = 2
    SG = NCH // NST
    assert SG % 2 == 0 and RPT % ZR == 0 and ZR <= K

    mesh = plsc.VectorSubcoreMesh(core_axis_name="c", subcore_axis_name="s")
    params = pltpu.CompilerParams(use_tc_tiling_on_sc=(D % 128 == 0))

    @functools.partial(
        pl.kernel,
        out_type=jax.ShapeDtypeStruct((NC, NPAD, D), jnp.float32),
        mesh=mesh,
        compiler_params=params,
        scratch_types=[
            pltpu.VMEM((SG, K), jnp.int32),
            pltpu.VMEM((SG, K), jnp.int32),
            pltpu.VMEM((K, D), jnp.float32),
            pltpu.VMEM((K, D), jnp.float32),
            pltpu.VMEM_SHARED((NPAD, D), jnp.float32),
            pltpu.SemaphoreType.DMA,
            pltpu.SemaphoreType.DMA,
            pltpu.SemaphoreType.DMA,
            pltpu.SemaphoreType.DMA,
        ],
    )
    def scat_k(y_hbm, src_hbm, dst_hbm, out_hbm,
               src_v, dst_v, rows_a, rows_b, acc_sh,
               sem_a, sem_b, sem_sa, sem_sb):
        c = lax.axis_index("c")
        s = lax.axis_index("s")
        wid = c * NS + s
        zero16 = jnp.zeros((L,), jnp.float32)

        @pl.loop(0, ZR)
        def _(r):
            for k in range(D // L):
                rows_a[r, pl.ds(k * L, L)] = zero16

        @pl.loop(0, RPT // ZR)
        def _(i):
            pltpu.sync_copy(rows_a.at[pl.ds(0, ZR)],
                            acc_sh.at[pl.ds(s * RPT + i * ZR, ZR)])

        plsc.subcore_barrier()

        UN = 8
        bufs = (rows_a, rows_b)
        gsems = (sem_a, sem_b)
        ssems = (sem_sa, sem_sb)
        for stage in range(NST):
            pltpu.sync_copy(src_hbm.at[wid, pl.ds(stage * SG, SG)], src_v)
            pltpu.sync_copy(dst_hbm.at[wid, pl.ds(stage * SG, SG)], dst_v)

            @pl.loop(0, SG // UN)
            def _(q):
                base = q * UN
                gd = [pltpu.async_copy(y_hbm.at[src_v.at[base]],
                                       rows_a, sem_a)]
                sd = [None, None]
                for i in range(1, UN + 1):
                    if i < UN:
                        if sd[i % 2] is not None:
                            sd[i % 2].wait()
                        gd.append(
                            pltpu.async_copy(y_hbm.at[src_v.at[base + i]],
                                             bufs[i % 2], gsems[i % 2]))
                    gd[i - 1].wait()
                    sd[(i - 1) % 2] = pltpu.async_copy(
                        bufs[(i - 1) % 2],
                        acc_sh.at[dst_v.at[base + i - 1]],
                        ssems[(i - 1) % 2], add=True)
                sd[0].wait()
                sd[1].wait()

        plsc.subcore_barrier()
        pltpu.sync_copy(acc_sh.at[pl.ds(s * RPT, RPT)],
                        out_hbm.at[c, pl.ds(s * RPT, RPT)])

    return scat_k


def _tc_matmul(x, W):
    N_, DI = x.shape
    DO = W.shape[1]

    def body(x_ref, w_ref, o_ref):
        o_ref[...] = jnp.dot(x_ref[...], w_ref[...],
                             preferred_element_type=jnp.float32)

    return pl.pallas_call(
        body,
        grid=(N_ // BN,),
        in_specs=[
            pl.BlockSpec((BN, DI), lambda i: (i, 0)),
            pl.BlockSpec((DI, DO), lambda i: (0, 0)),
        ],
        out_specs=pl.BlockSpec((BN, DO), lambda i: (i, 0)),
        out_shape=jax.ShapeDtypeStruct((N_, DO), jnp.float32),
    )(x, W)


def _tc_dinv_scale(degp_t, xw):
    N_, D = xw.shape

    def body(dp_ref, xw_ref, y_ref, dinv_ref):
        deg = jnp.sum(dp_ref[...], axis=1, keepdims=True) + 1.0
        dinv = lax.rsqrt(jnp.maximum(deg, 1e-12))
        dinv_ref[...] = dinv
        y_ref[...] = xw_ref[...] * dinv

    return pl.pallas_call(
        body,
        grid=(N_ // BN,),
        in_specs=[
            pl.BlockSpec((BN, NC), lambda i: (i, 0)),
            pl.BlockSpec((BN, D), lambda i: (i, 0)),
        ],
        out_specs=[
            pl.BlockSpec((BN, D), lambda i: (i, 0)),
            pl.BlockSpec((BN, 1), lambda i: (i, 0)),
        ],
        out_shape=[
            jax.ShapeDtypeStruct((N_, D), jnp.float32),
            jax.ShapeDtypeStruct((N_, 1), jnp.float32),
        ],
    )(degp_t, xw)


def _tc_combine_matmul(parts, y, dinv, b, W, scale_out):
    N_, D = y.shape
    DO = W.shape[1]

    def body(p_ref, y_ref, dinv_ref, b_ref, w_ref, o_ref):
        S = p_ref[0] + p_ref[1] + y_ref[...]
        h = jnp.maximum(S * dinv_ref[...] + b_ref[...], 0.0)
        o = jnp.dot(h, w_ref[...], preferred_element_type=jnp.float32)
        if scale_out:
            o = o * dinv_ref[...]
        o_ref[...] = o

    return pl.pallas_call(
        body,
        grid=(N_ // BN,),
        in_specs=[
            pl.BlockSpec((NC, BN, D), lambda i: (0, i, 0)),
            pl.BlockSpec((BN, D), lambda i: (i, 0)),
            pl.BlockSpec((BN, 1), lambda i: (i, 0)),
            pl.BlockSpec((1, D), lambda i: (0, 0)),
            pl.BlockSpec((D, DO), lambda i: (0, 0)),
        ],
        out_specs=pl.BlockSpec((BN, DO), lambda i: (i, 0)),
        out_shape=jax.ShapeDtypeStruct((N_, DO), jnp.float32),
    )(parts, y, dinv, b, W)


def _tc_final(parts, y, dinv, b, W, b_out):
    N_, D = y.shape
    DO = W.shape[1]

    def body(p_ref, y_ref, dinv_ref, b_ref, w_ref, bo_ref, o_ref):
        S = p_ref[0] + p_ref[1] + y_ref[...]
        h = jnp.maximum(S * dinv_ref[...] + b_ref[...], 0.0)
        o_ref[...] = jnp.dot(h, w_ref[...],
                             preferred_element_type=jnp.float32) + bo_ref[...]

    return pl.pallas_call(
        body,
        grid=(N_ // BN,),
        in_specs=[
            pl.BlockSpec((NC, BN, D), lambda i: (0, i, 0)),
            pl.BlockSpec((BN, D), lambda i: (i, 0)),
            pl.BlockSpec((BN, 1), lambda i: (i, 0)),
            pl.BlockSpec((1, D), lambda i: (0, 0)),
            pl.BlockSpec((D, DO), lambda i: (0, 0)),
            pl.BlockSpec((1, DO), lambda i: (0, 0)),
        ],
        out_specs=pl.BlockSpec((BN, DO), lambda i: (i, 0)),
        out_shape=jax.ShapeDtypeStruct((N_, DO), jnp.float32),
    )(parts, y, dinv, b, W, b_out)


def kernel(x, edge_index, W1, b1, W2, b2, W_out, b_out):
    N_, D_in = x.shape
    E = edge_index.shape[1]
    assert E % NW == 0
    NPAD = ((N_ + NS * L - 1) // (NS * L)) * (NS * L)

    EP = E // NW
    EPP = ((EP + 2 * KC - 1) // (2 * KC)) * (2 * KC)
    NCH = EPP // KC
    src3 = jnp.pad(edge_index[0].reshape(NW, EP), ((0, 0), (0, EPP - EP)),
                   constant_values=0).reshape(NW, NCH, KC)
    pad_dst = N_ + jnp.arange(EPP - EP, dtype=jnp.int32) % (NPAD - N_)
    dst3 = jnp.concatenate(
        [edge_index[1].reshape(NW, EP),
         jnp.broadcast_to(pad_dst, (NW, EPP - EP))], axis=1,
    ).reshape(NW, NCH, KC)

    degp = _deg_kernel(NCH, NPAD)(dst3)
    xw1 = _tc_matmul(x, W1)
    degp_t = degp.T

    y1, dinv = _tc_dinv_scale(degp_t, xw1)

    parts1 = _scatter_kernel(NCH, NPAD, y1.shape[1])(y1, src3, dst3)

    y2 = _tc_combine_matmul(parts1, y1, dinv, b1.reshape(1, -1), W2, True)

    parts2 = _scatter_kernel(NCH, NPAD, y2.shape[1])(y2, src3, dst3)

    return _tc_final(parts2, y2, dinv, b2.reshape(1, -1), W_out,
                     b_out.reshape(1, -1))


# speedup vs baseline: 1.0172x; 1.0172x over previous
import functools

import jax
import jax.numpy as jnp
from jax import lax
from jax.experimental import pallas as pl
from jax.experimental.pallas import tpu as pltpu
from jax.experimental.pallas import tpu_sc as plsc

NC = 2
NS = 16
NW = NC * NS
L = 16

BN = 2000
KC = 125


@functools.lru_cache(maxsize=None)
def _deg_kernel(NCH, NPAD):
    K = KC
    RPT = NPAD // NS

    mesh = plsc.VectorSubcoreMesh(core_axis_name="c", subcore_axis_name="s")

    @functools.partial(
        pl.kernel,
        out_type=jax.ShapeDtypeStruct((NC, NPAD), jnp.float32),
        mesh=mesh,
        scratch_types=[
            pltpu.VMEM((NCH, K), jnp.int32),
            pltpu.VMEM((128,), jnp.float32),
            pltpu.VMEM((RPT,), jnp.float32),
            pltpu.VMEM_SHARED((NPAD,), jnp.float32),
        ],
    )
    def deg_k(dst_hbm, out_hbm, dst_v, ones_v, z_v, acc_sh):
        c = lax.axis_index("c")
        s = lax.axis_index("s")
        wid = c * NS + s
        one16 = jnp.full((L,), 1.0, jnp.float32)
        zero16 = jnp.zeros((L,), jnp.float32)
        for k in range(128 // L):
            ones_v[pl.ds(k * L, L)] = one16
        for k in range(RPT // L):
            z_v[pl.ds(k * L, L)] = zero16
        pltpu.sync_copy(z_v, acc_sh.at[pl.ds(s * RPT, RPT)])
        pltpu.sync_copy(dst_hbm.at[wid], dst_v)
        plsc.subcore_barrier()

        @pl.loop(0, NCH)
        def _(j):
            pltpu.sync_copy(ones_v.at[pl.ds(0, K)], acc_sh.at[dst_v.at[j]],
                            add=True)

        plsc.subcore_barrier()
        pltpu.sync_copy(acc_sh.at[pl.ds(s * RPT, RPT)],
                        out_hbm.at[c, pl.ds(s * RPT, RPT)])

    return deg_k


@functools.lru_cache(maxsize=None)
def _scatter_kernel(NCH, NPAD, D):
    K = KC
    RPT = NPAD // NS
    ZR = 32
    NST = 2
    SG = NCH // NST
    assert SG % 2 == 0 and RPT % ZR == 0 and ZR <= K

    mesh = plsc.VectorSubcoreMesh(core_axis_name="c", subcore_axis_name="s")
    params = pltpu.CompilerParams(use_tc_tiling_on_sc=(D % 128 == 0))

    @functools.partial(
        pl.kernel,
        out_type=jax.ShapeDtypeStruct((NC, NPAD, D), jnp.float32),
        mesh=mesh,
        compiler_params=params,
        scratch_types=[
            pltpu.VMEM((SG, K), jnp.int32),
            pltpu.VMEM((SG, K), jnp.int32),
            pltpu.VMEM((K, D), jnp.float32),
            pltpu.VMEM((K, D), jnp.float32),
            pltpu.VMEM_SHARED((NPAD, D), jnp.float32),
            pltpu.SemaphoreType.DMA,
            pltpu.SemaphoreType.DMA,
            pltpu.SemaphoreType.DMA,
            pltpu.SemaphoreType.DMA,
        ],
    )
    def scat_k(y_hbm, src_hbm, dst_hbm, out_hbm,
               src_v, dst_v, rows_a, rows_b, acc_sh,
               sem_a, sem_b, sem_sa, sem_sb):
        c = lax.axis_index("c")
        s = lax.axis_index("s")
        wid = c * NS + s
        zero16 = jnp.zeros((L,), jnp.float32)

        @pl.loop(0, ZR)
        def _(r):
            for k in range(D // L):
                rows_a[r, pl.ds(k * L, L)] = zero16

        zd = [pltpu.async_copy(rows_a.at[pl.ds(0, ZR)],
                               acc_sh.at[pl.ds(s * RPT + i * ZR, ZR)],
                               sem_sa)
              for i in range(RPT // ZR)]
        pltpu.sync_copy(src_hbm.at[wid, pl.ds(0, SG)], src_v)
        pltpu.sync_copy(dst_hbm.at[wid, pl.ds(0, SG)], dst_v)
        for d in zd:
            d.wait()
        plsc.subcore_barrier()

        UN = 8
        bufs = (rows_a, rows_b)
        gsems = (sem_a, sem_b)
        ssems = (sem_sa, sem_sb)
        for stage in range(NST):
            if stage > 0:
                pltpu.sync_copy(src_hbm.at[wid, pl.ds(stage * SG, SG)], src_v)
                pltpu.sync_copy(dst_hbm.at[wid, pl.ds(stage * SG, SG)], dst_v)

            @pl.loop(0, SG // UN)
            def _(q):
                base = q * UN
                gd = [pltpu.async_copy(y_hbm.at[src_v.at[base]],
                                       rows_a, sem_a)]
                sd = [None, None]
                for i in range(1, UN + 1):
                    if i < UN:
                        if sd[i % 2] is not None:
                            sd[i % 2].wait()
                        gd.append(
                            pltpu.async_copy(y_hbm.at[src_v.at[base + i]],
                                             bufs[i % 2], gsems[i % 2]))
                    gd[i - 1].wait()
                    sd[(i - 1) % 2] = pltpu.async_copy(
                        bufs[(i - 1) % 2],
                        acc_sh.at[dst_v.at[base + i - 1]],
                        ssems[(i - 1) % 2], add=True)
                sd[0].wait()
                sd[1].wait()

        plsc.subcore_barrier()
        pltpu.sync_copy(acc_sh.at[pl.ds(s * RPT, RPT)],
                        out_hbm.at[c, pl.ds(s * RPT, RPT)])

    return scat_k


def _tc_matmul(x, W):
    N_, DI = x.shape
    DO = W.shape[1]

    def body(x_ref, w_ref, o_ref):
        o_ref[...] = jnp.dot(x_ref[...], w_ref[...],
                             preferred_element_type=jnp.float32)

    return pl.pallas_call(
        body,
        grid=(N_ // BN,),
        in_specs=[
            pl.BlockSpec((BN, DI), lambda i: (i, 0)),
            pl.BlockSpec((DI, DO), lambda i: (0, 0)),
        ],
        out_specs=pl.BlockSpec((BN, DO), lambda i: (i, 0)),
        out_shape=jax.ShapeDtypeStruct((N_, DO), jnp.float32),
    )(x, W)


def _tc_dinv_scale(degp_t, xw):
    N_, D = xw.shape

    def body(dp_ref, xw_ref, y_ref, dinv_ref):
        deg = jnp.sum(dp_ref[...], axis=1, keepdims=True) + 1.0
        dinv = lax.rsqrt(jnp.maximum(deg, 1e-12))
        dinv_ref[...] = dinv
        y_ref[...] = xw_ref[...] * dinv

    return pl.pallas_call(
        body,
        grid=(N_ // BN,),
        in_specs=[
            pl.BlockSpec((BN, NC), lambda i: (i, 0)),
            pl.BlockSpec((BN, D), lambda i: (i, 0)),
        ],
        out_specs=[
            pl.BlockSpec((BN, D), lambda i: (i, 0)),
            pl.BlockSpec((BN, 1), lambda i: (i, 0)),
        ],
        out_shape=[
            jax.ShapeDtypeStruct((N_, D), jnp.float32),
            jax.ShapeDtypeStruct((N_, 1), jnp.float32),
        ],
    )(degp_t, xw)


def _tc_combine_matmul(parts, y, dinv, b, W, scale_out):
    N_, D = y.shape
    DO = W.shape[1]

    def body(p_ref, y_ref, dinv_ref, b_ref, w_ref, o_ref):
        S = p_ref[0] + p_ref[1] + y_ref[...]
        h = jnp.maximum(S * dinv_ref[...] + b_ref[...], 0.0)
        o = jnp.dot(h, w_ref[...], preferred_element_type=jnp.float32)
        if scale_out:
            o = o * dinv_ref[...]
        o_ref[...] = o

    return pl.pallas_call(
        body,
        grid=(N_ // BN,),
        in_specs=[
            pl.BlockSpec((NC, BN, D), lambda i: (0, i, 0)),
            pl.BlockSpec((BN, D), lambda i: (i, 0)),
            pl.BlockSpec((BN, 1), lambda i: (i, 0)),
            pl.BlockSpec((1, D), lambda i: (0, 0)),
            pl.BlockSpec((D, DO), lambda i: (0, 0)),
        ],
        out_specs=pl.BlockSpec((BN, DO), lambda i: (i, 0)),
        out_shape=jax.ShapeDtypeStruct((N_, DO), jnp.float32),
    )(parts, y, dinv, b, W)


def _tc_final(parts, y, dinv, b, W, b_out):
    N_, D = y.shape
    DO = W.shape[1]

    def body(p_ref, y_ref, dinv_ref, b_ref, w_ref, bo_ref, o_ref):
        S = p_ref[0] + p_ref[1] + y_ref[...]
        h = jnp.maximum(S * dinv_ref[...] + b_ref[...], 0.0)
        o_ref[...] = jnp.dot(h, w_ref[...],
                             preferred_element_type=jnp.float32) + bo_ref[...]

    return pl.pallas_call(
        body,
        grid=(N_ // BN,),
        in_specs=[
            pl.BlockSpec((NC, BN, D), lambda i: (0, i, 0)),
            pl.BlockSpec((BN, D), lambda i: (i, 0)),
            pl.BlockSpec((BN, 1), lambda i: (i, 0)),
            pl.BlockSpec((1, D), lambda i: (0, 0)),
            pl.BlockSpec((D, DO), lambda i: (0, 0)),
            pl.BlockSpec((1, DO), lambda i: (0, 0)),
        ],
        out_specs=pl.BlockSpec((BN, DO), lambda i: (i, 0)),
        out_shape=jax.ShapeDtypeStruct((N_, DO), jnp.float32),
    )(parts, y, dinv, b, W, b_out)


def kernel(x, edge_index, W1, b1, W2, b2, W_out, b_out):
    N_, D_in = x.shape
    E = edge_index.shape[1]
    assert E % NW == 0
    NPAD = ((N_ + NS * L - 1) // (NS * L)) * (NS * L)

    EP = E // NW
    EPP = ((EP + 2 * KC - 1) // (2 * KC)) * (2 * KC)
    NCH = EPP // KC
    src3 = jnp.pad(edge_index[0].reshape(NW, EP), ((0, 0), (0, EPP - EP)),
                   constant_values=0).reshape(NW, NCH, KC)
    pad_dst = N_ + jnp.arange(EPP - EP, dtype=jnp.int32) % (NPAD - N_)
    dst3 = jnp.concatenate(
        [edge_index[1].reshape(NW, EP),
         jnp.broadcast_to(pad_dst, (NW, EPP - EP))], axis=1,
    ).reshape(NW, NCH, KC)

    degp = _deg_kernel(NCH, NPAD)(dst3)
    xw1 = _tc_matmul(x, W1)
    degp_t = degp.T

    y1, dinv = _tc_dinv_scale(degp_t, xw1)

    parts1 = _scatter_kernel(NCH, NPAD, y1.shape[1])(y1, src3, dst3)

    y2 = _tc_combine_matmul(parts1, y1, dinv, b1.reshape(1, -1), W2, True)

    parts2 = _scatter_kernel(NCH, NPAD, y2.shape[1])(y2, src3, dst3)

    return _tc_final(parts2, y2, dinv, b2.reshape(1, -1), W_out,
                     b_out.reshape(1, -1))


# speedup vs baseline: 1.0295x; 1.0120x over previous
import functools

import jax
import jax.numpy as jnp
from jax import lax
from jax.experimental import pallas as pl
from jax.experimental.pallas import tpu as pltpu
from jax.experimental.pallas import tpu_sc as plsc

NC = 2
NS = 16
NW = NC * NS
L = 16

BN = 2000
KC = 125


@functools.lru_cache(maxsize=None)
def _deg_kernel(NCH, NPAD):
    K = KC
    RPT = NPAD // NS

    mesh = plsc.VectorSubcoreMesh(core_axis_name="c", subcore_axis_name="s")

    @functools.partial(
        pl.kernel,
        out_type=jax.ShapeDtypeStruct((NC, NPAD), jnp.float32),
        mesh=mesh,
        scratch_types=[
            pltpu.VMEM((NCH, K), jnp.int32),
            pltpu.VMEM((128,), jnp.float32),
            pltpu.VMEM((RPT,), jnp.float32),
            pltpu.VMEM_SHARED((NPAD,), jnp.float32),
        ],
    )
    def deg_k(dst_hbm, out_hbm, dst_v, ones_v, z_v, acc_sh):
        c = lax.axis_index("c")
        s = lax.axis_index("s")
        wid = c * NS + s
        one16 = jnp.full((L,), 1.0, jnp.float32)
        zero16 = jnp.zeros((L,), jnp.float32)
        for k in range(128 // L):
            ones_v[pl.ds(k * L, L)] = one16
        for k in range(RPT // L):
            z_v[pl.ds(k * L, L)] = zero16
        pltpu.sync_copy(z_v, acc_sh.at[pl.ds(s * RPT, RPT)])
        pltpu.sync_copy(dst_hbm.at[wid], dst_v)
        plsc.subcore_barrier()

        @pl.loop(0, NCH)
        def _(j):
            pltpu.sync_copy(ones_v.at[pl.ds(0, K)], acc_sh.at[dst_v.at[j]],
                            add=True)

        plsc.subcore_barrier()
        pltpu.sync_copy(acc_sh.at[pl.ds(s * RPT, RPT)],
                        out_hbm.at[c, pl.ds(s * RPT, RPT)])

    return deg_k


@functools.lru_cache(maxsize=None)
def _scatter_kernel(NCH, NPAD, D):
    K = KC
    RPT = NPAD // NS
    ZR = 32
    NST = 2
    SG = NCH // NST
    assert SG % 2 == 0 and RPT % ZR == 0 and ZR <= K

    mesh = plsc.VectorSubcoreMesh(core_axis_name="c", subcore_axis_name="s")
    params = pltpu.CompilerParams(use_tc_tiling_on_sc=(D % 128 == 0))

    @functools.partial(
        pl.kernel,
        out_type=jax.ShapeDtypeStruct((NC, NPAD, D), jnp.float32),
        mesh=mesh,
        compiler_params=params,
        scratch_types=[
            pltpu.VMEM((SG, K), jnp.int32),
            pltpu.VMEM((SG, K), jnp.int32),
            pltpu.VMEM((K, D), jnp.float32),
            pltpu.VMEM((K, D), jnp.float32),
            pltpu.VMEM_SHARED((NPAD, D), jnp.float32),
            pltpu.SemaphoreType.DMA,
            pltpu.SemaphoreType.DMA,
            pltpu.SemaphoreType.DMA,
            pltpu.SemaphoreType.DMA,
        ],
    )
    def scat_k(y_hbm, src_hbm, dst_hbm, out_hbm,
               src_v, dst_v, rows_a, rows_b, acc_sh,
               sem_a, sem_b, sem_sa, sem_sb):
        c = lax.axis_index("c")
        s = lax.axis_index("s")
        wid = c * NS + s
        zero16 = jnp.zeros((L,), jnp.float32)

        @pl.loop(0, ZR)
        def _(r):
            for k in range(D // L):
                rows_a[r, pl.ds(k * L, L)] = zero16

        zd = [pltpu.async_copy(rows_a.at[pl.ds(0, ZR)],
                               acc_sh.at[pl.ds(s * RPT + i * ZR, ZR)],
                               sem_sa)
              for i in range(RPT // ZR)]
        pltpu.sync_copy(src_hbm.at[wid, pl.ds(0, SG)], src_v)
        pltpu.sync_copy(dst_hbm.at[wid, pl.ds(0, SG)], dst_v)
        for d in zd:
            d.wait()
        plsc.subcore_barrier()

        UN = 10
        bufs = (rows_a, rows_b)
        gsems = (sem_a, sem_b)
        ssems = (sem_sa, sem_sb)
        for stage in range(NST):
            if stage > 0:
                pltpu.sync_copy(src_hbm.at[wid, pl.ds(stage * SG, SG)], src_v)
                pltpu.sync_copy(dst_hbm.at[wid, pl.ds(stage * SG, SG)], dst_v)

            @pl.loop(0, SG // UN)
            def _(q):
                base = q * UN
                gd = [pltpu.async_copy(y_hbm.at[src_v.at[base]],
                                       rows_a, sem_a)]
                sd = [None, None]
                for i in range(1, UN + 1):
                    if i < UN:
                        if sd[i % 2] is not None:
                            sd[i % 2].wait()
                        gd.append(
                            pltpu.async_copy(y_hbm.at[src_v.at[base + i]],
                                             bufs[i % 2], gsems[i % 2]))
                    gd[i - 1].wait()
                    sd[(i - 1) % 2] = pltpu.async_copy(
                        bufs[(i - 1) % 2],
                        acc_sh.at[dst_v.at[base + i - 1]],
                        ssems[(i - 1) % 2], add=True)
                sd[0].wait()
                sd[1].wait()

        plsc.subcore_barrier()
        pltpu.sync_copy(acc_sh.at[pl.ds(s * RPT, RPT)],
                        out_hbm.at[c, pl.ds(s * RPT, RPT)])

    return scat_k


def _tc_dinv_scale(degp_t, x, W):
    N_, DI = x.shape
    D = W.shape[1]

    def body(dp_ref, x_ref, w_ref, y_ref, dinv_ref):
        deg = jnp.sum(dp_ref[...], axis=1, keepdims=True) + 1.0
        dinv = lax.rsqrt(jnp.maximum(deg, 1e-12))
        dinv_ref[...] = dinv
        xw = jnp.dot(x_ref[...], w_ref[...],
                     preferred_element_type=jnp.float32)
        y_ref[...] = xw * dinv

    return pl.pallas_call(
        body,
        grid=(N_ // BN,),
        in_specs=[
            pl.BlockSpec((BN, NC), lambda i: (i, 0)),
            pl.BlockSpec((BN, DI), lambda i: (i, 0)),
            pl.BlockSpec((DI, D), lambda i: (0, 0)),
        ],
        out_specs=[
            pl.BlockSpec((BN, D), lambda i: (i, 0)),
            pl.BlockSpec((BN, 1), lambda i: (i, 0)),
        ],
        out_shape=[
            jax.ShapeDtypeStruct((N_, D), jnp.float32),
            jax.ShapeDtypeStruct((N_, 1), jnp.float32),
        ],
    )(degp_t, x, W)


def _tc_combine_matmul(parts, y, dinv, b, W, scale_out):
    N_, D = y.shape
    DO = W.shape[1]

    def body(p_ref, y_ref, dinv_ref, b_ref, w_ref, o_ref):
        S = p_ref[0] + p_ref[1] + y_ref[...]
        h = jnp.maximum(S * dinv_ref[...] + b_ref[...], 0.0)
        o = jnp.dot(h, w_ref[...], preferred_element_type=jnp.float32)
        if scale_out:
            o = o * dinv_ref[...]
        o_ref[...] = o

    return pl.pallas_call(
        body,
        grid=(N_ // BN,),
        in_specs=[
            pl.BlockSpec((NC, BN, D), lambda i: (0, i, 0)),
            pl.BlockSpec((BN, D), lambda i: (i, 0)),
            pl.BlockSpec((BN, 1), lambda i: (i, 0)),
            pl.BlockSpec((1, D), lambda i: (0, 0)),
            pl.BlockSpec((D, DO), lambda i: (0, 0)),
        ],
        out_specs=pl.BlockSpec((BN, DO), lambda i: (i, 0)),
        out_shape=jax.ShapeDtypeStruct((N_, DO), jnp.float32),
    )(parts, y, dinv, b, W)


def _tc_final(parts, y, dinv, b, W, b_out):
    N_, D = y.shape
    DO = W.shape[1]

    def body(p_ref, y_ref, dinv_ref, b_ref, w_ref, bo_ref, o_ref):
        S = p_ref[0] + p_ref[1] + y_ref[...]
        h = jnp.maximum(S * dinv_ref[...] + b_ref[...], 0.0)
        o_ref[...] = jnp.dot(h, w_ref[...],
                             preferred_element_type=jnp.float32) + bo_ref[...]

    return pl.pallas_call(
        body,
        grid=(N_ // BN,),
        in_specs=[
            pl.BlockSpec((NC, BN, D), lambda i: (0, i, 0)),
            pl.BlockSpec((BN, D), lambda i: (i, 0)),
            pl.BlockSpec((BN, 1), lambda i: (i, 0)),
            pl.BlockSpec((1, D), lambda i: (0, 0)),
            pl.BlockSpec((D, DO), lambda i: (0, 0)),
            pl.BlockSpec((1, DO), lambda i: (0, 0)),
        ],
        out_specs=pl.BlockSpec((BN, DO), lambda i: (i, 0)),
        out_shape=jax.ShapeDtypeStruct((N_, DO), jnp.float32),
    )(parts, y, dinv, b, W, b_out)


def kernel(x, edge_index, W1, b1, W2, b2, W_out, b_out):
    N_, D_in = x.shape
    E = edge_index.shape[1]
    assert E % NW == 0
    NPAD = ((N_ + NS * L - 1) // (NS * L)) * (NS * L)

    EP = E // NW
    EPP = ((EP + 2 * KC - 1) // (2 * KC)) * (2 * KC)
    NCH = EPP // KC
    src3 = jnp.pad(edge_index[0].reshape(NW, EP), ((0, 0), (0, EPP - EP)),
                   constant_values=0).reshape(NW, NCH, KC)
    pad_dst = N_ + jnp.arange(EPP - EP, dtype=jnp.int32) % (NPAD - N_)
    dst3 = jnp.concatenate(
        [edge_index[1].reshape(NW, EP),
         jnp.broadcast_to(pad_dst, (NW, EPP - EP))], axis=1,
    ).reshape(NW, NCH, KC)

    degp = _deg_kernel(NCH, NPAD)(dst3)
    degp_t = degp.T

    y1, dinv = _tc_dinv_scale(degp_t, x, W1)

    parts1 = _scatter_kernel(NCH, NPAD, y1.shape[1])(y1, src3, dst3)

    y2 = _tc_combine_matmul(parts1, y1, dinv, b1.reshape(1, -1), W2, True)

    parts2 = _scatter_kernel(NCH, NPAD, y2.shape[1])(y2, src3, dst3)

    return _tc_final(parts2, y2, dinv, b2.reshape(1, -1), W_out,
                     b_out.reshape(1, -1))


# speedup vs baseline: 1.0469x; 1.0169x over previous
import functools

import jax
import jax.numpy as jnp
from jax import lax
from jax.experimental import pallas as pl
from jax.experimental.pallas import tpu as pltpu
from jax.experimental.pallas import tpu_sc as plsc

NC = 2
NS = 16
NW = NC * NS
L = 16

BN = 2000
KC = 125


@functools.lru_cache(maxsize=None)
def _deg_kernel(NCH, NPAD):
    K = KC
    RPT = NPAD // NS

    mesh = plsc.VectorSubcoreMesh(core_axis_name="c", subcore_axis_name="s")

    @functools.partial(
        pl.kernel,
        out_type=jax.ShapeDtypeStruct((NC, NPAD), jnp.float32),
        mesh=mesh,
        scratch_types=[
            pltpu.VMEM((NCH, K), jnp.int32),
            pltpu.VMEM((128,), jnp.float32),
            pltpu.VMEM((RPT,), jnp.float32),
            pltpu.VMEM_SHARED((NPAD,), jnp.float32),
            pltpu.SemaphoreType.DMA,
        ],
    )
    def deg_k(dst_hbm, out_hbm, dst_v, ones_v, z_v, acc_sh, sem_d):
        c = lax.axis_index("c")
        s = lax.axis_index("s")
        wid = c * NS + s
        one16 = jnp.full((L,), 1.0, jnp.float32)
        zero16 = jnp.zeros((L,), jnp.float32)
        for k in range(128 // L):
            ones_v[pl.ds(k * L, L)] = one16
        for k in range(RPT // L):
            z_v[pl.ds(k * L, L)] = zero16
        pltpu.sync_copy(z_v, acc_sh.at[pl.ds(s * RPT, RPT)])
        pltpu.sync_copy(dst_hbm.at[wid], dst_v)
        plsc.subcore_barrier()

        UN = 16
        assert NCH % UN == 0

        @pl.loop(0, NCH // UN)
        def _(q):
            base = q * UN
            ds_ = [pltpu.async_copy(ones_v.at[pl.ds(0, K)],
                                    acc_sh.at[dst_v.at[base + i]],
                                    sem_d, add=True)
                   for i in range(UN)]
            for d in ds_:
                d.wait()

        plsc.subcore_barrier()
        pltpu.sync_copy(acc_sh.at[pl.ds(s * RPT, RPT)],
                        out_hbm.at[c, pl.ds(s * RPT, RPT)])

    return deg_k


@functools.lru_cache(maxsize=None)
def _scatter_kernel(NCH, NPAD, D):
    K = KC
    RPT = NPAD // NS
    ZR = 32
    NST = 2
    SG = NCH // NST
    assert SG % 2 == 0 and RPT % ZR == 0 and ZR <= K

    mesh = plsc.VectorSubcoreMesh(core_axis_name="c", subcore_axis_name="s")
    params = pltpu.CompilerParams(use_tc_tiling_on_sc=(D % 128 == 0))

    @functools.partial(
        pl.kernel,
        out_type=jax.ShapeDtypeStruct((NC, NPAD, D), jnp.float32),
        mesh=mesh,
        compiler_params=params,
        scratch_types=[
            pltpu.VMEM((SG, K), jnp.int32),
            pltpu.VMEM((SG, K), jnp.int32),
            pltpu.VMEM((K, D), jnp.float32),
            pltpu.VMEM((K, D), jnp.float32),
            pltpu.VMEM_SHARED((NPAD, D), jnp.float32),
            pltpu.SemaphoreType.DMA,
            pltpu.SemaphoreType.DMA,
            pltpu.SemaphoreType.DMA,
            pltpu.SemaphoreType.DMA,
        ],
    )
    def scat_k(y_hbm, src_hbm, dst_hbm, out_hbm,
               src_v, dst_v, rows_a, rows_b, acc_sh,
               sem_a, sem_b, sem_sa, sem_sb):
        c = lax.axis_index("c")
        s = lax.axis_index("s")
        wid = c * NS + s
        zero16 = jnp.zeros((L,), jnp.float32)

        @pl.loop(0, ZR)
        def _(r):
            for k in range(D // L):
                rows_a[r, pl.ds(k * L, L)] = zero16

        zd = [pltpu.async_copy(rows_a.at[pl.ds(0, ZR)],
                               acc_sh.at[pl.ds(s * RPT + i * ZR, ZR)],
                               sem_sa)
              for i in range(RPT // ZR)]
        pltpu.sync_copy(src_hbm.at[wid, pl.ds(0, SG)], src_v)
        pltpu.sync_copy(dst_hbm.at[wid, pl.ds(0, SG)], dst_v)
        for d in zd:
            d.wait()
        plsc.subcore_barrier()

        UN = 10
        bufs = (rows_a, rows_b)
        gsems = (sem_a, sem_b)
        ssems = (sem_sa, sem_sb)
        for stage in range(NST):
            if stage > 0:
                pltpu.sync_copy(src_hbm.at[wid, pl.ds(stage * SG, SG)], src_v)
                pltpu.sync_copy(dst_hbm.at[wid, pl.ds(stage * SG, SG)], dst_v)

            @pl.loop(0, SG // UN)
            def _(q):
                base = q * UN
                gd = [pltpu.async_copy(y_hbm.at[src_v.at[base]],
                                       rows_a, sem_a)]
                sd = [None, None]
                for i in range(1, UN + 1):
                    if i < UN:
                        if sd[i % 2] is not None:
                            sd[i % 2].wait()
                        gd.append(
                            pltpu.async_copy(y_hbm.at[src_v.at[base + i]],
                                             bufs[i % 2], gsems[i % 2]))
                    gd[i - 1].wait()
                    sd[(i - 1) % 2] = pltpu.async_copy(
                        bufs[(i - 1) % 2],
                        acc_sh.at[dst_v.at[base + i - 1]],
                        ssems[(i - 1) % 2], add=True)
                sd[0].wait()
                sd[1].wait()

        plsc.subcore_barrier()
        pltpu.sync_copy(acc_sh.at[pl.ds(s * RPT, RPT)],
                        out_hbm.at[c, pl.ds(s * RPT, RPT)])

    return scat_k


def _tc_dinv_scale(degp_t, x, W):
    N_, DI = x.shape
    D = W.shape[1]

    def body(dp_ref, x_ref, w_ref, y_ref, dinv_ref):
        deg = jnp.sum(dp_ref[...], axis=1, keepdims=True) + 1.0
        dinv = lax.rsqrt(jnp.maximum(deg, 1e-12))
        dinv_ref[...] = dinv
        xw = jnp.dot(x_ref[...], w_ref[...],
                     preferred_element_type=jnp.float32)
        y_ref[...] = xw * dinv

    return pl.pallas_call(
        body,
        grid=(N_ // BN,),
        in_specs=[
            pl.BlockSpec((BN, NC), lambda i: (i, 0)),
            pl.BlockSpec((BN, DI), lambda i: (i, 0)),
            pl.BlockSpec((DI, D), lambda i: (0, 0)),
        ],
        out_specs=[
            pl.BlockSpec((BN, D), lambda i: (i, 0)),
            pl.BlockSpec((BN, 1), lambda i: (i, 0)),
        ],
        out_shape=[
            jax.ShapeDtypeStruct((N_, D), jnp.float32),
            jax.ShapeDtypeStruct((N_, 1), jnp.float32),
        ],
    )(degp_t, x, W)


def _tc_combine_matmul(parts, y, dinv, b, W, scale_out):
    N_, D = y.shape
    DO = W.shape[1]

    def body(p_ref, y_ref, dinv_ref, b_ref, w_ref, o_ref):
        S = p_ref[0] + p_ref[1] + y_ref[...]
        h = jnp.maximum(S * dinv_ref[...] + b_ref[...], 0.0)
        o = jnp.dot(h, w_ref[...], preferred_element_type=jnp.float32)
        if scale_out:
            o = o * dinv_ref[...]
        o_ref[...] = o

    return pl.pallas_call(
        body,
        grid=(N_ // BN,),
        in_specs=[
            pl.BlockSpec((NC, BN, D), lambda i: (0, i, 0)),
            pl.BlockSpec((BN, D), lambda i: (i, 0)),
            pl.BlockSpec((BN, 1), lambda i: (i, 0)),
            pl.BlockSpec((1, D), lambda i: (0, 0)),
            pl.BlockSpec((D, DO), lambda i: (0, 0)),
        ],
        out_specs=pl.BlockSpec((BN, DO), lambda i: (i, 0)),
        out_shape=jax.ShapeDtypeStruct((N_, DO), jnp.float32),
    )(parts, y, dinv, b, W)


def _tc_final(parts, y, dinv, b, W, b_out):
    N_, D = y.shape
    DO = W.shape[1]

    def body(p_ref, y_ref, dinv_ref, b_ref, w_ref, bo_ref, o_ref):
        S = p_ref[0] + p_ref[1] + y_ref[...]
        h = jnp.maximum(S * dinv_ref[...] + b_ref[...], 0.0)
        o_ref[...] = jnp.dot(h, w_ref[...],
                             preferred_element_type=jnp.float32) + bo_ref[...]

    return pl.pallas_call(
        body,
        grid=(N_ // BN,),
        in_specs=[
            pl.BlockSpec((NC, BN, D), lambda i: (0, i, 0)),
            pl.BlockSpec((BN, D), lambda i: (i, 0)),
            pl.BlockSpec((BN, 1), lambda i: (i, 0)),
            pl.BlockSpec((1, D), lambda i: (0, 0)),
            pl.BlockSpec((D, DO), lambda i: (0, 0)),
            pl.BlockSpec((1, DO), lambda i: (0, 0)),
        ],
        out_specs=pl.BlockSpec((BN, DO), lambda i: (i, 0)),
        out_shape=jax.ShapeDtypeStruct((N_, DO), jnp.float32),
    )(parts, y, dinv, b, W, b_out)


def kernel(x, edge_index, W1, b1, W2, b2, W_out, b_out):
    N_, D_in = x.shape
    E = edge_index.shape[1]
    assert E % NW == 0
    NPAD = ((N_ + NS * L - 1) // (NS * L)) * (NS * L)

    EP = E // NW
    EPP = ((EP + 2 * KC - 1) // (2 * KC)) * (2 * KC)
    NCH = EPP // KC
    src3 = jnp.pad(edge_index[0].reshape(NW, EP), ((0, 0), (0, EPP - EP)),
                   constant_values=0).reshape(NW, NCH, KC)
    pad_dst = N_ + jnp.arange(EPP - EP, dtype=jnp.int32) % (NPAD - N_)
    dst3 = jnp.concatenate(
        [edge_index[1].reshape(NW, EP),
         jnp.broadcast_to(pad_dst, (NW, EPP - EP))], axis=1,
    ).reshape(NW, NCH, KC)

    degp = _deg_kernel(NCH, NPAD)(dst3)
    degp_t = degp.T

    y1, dinv = _tc_dinv_scale(degp_t, x, W1)

    parts1 = _scatter_kernel(NCH, NPAD, y1.shape[1])(y1, src3, dst3)

    y2 = _tc_combine_matmul(parts1, y1, dinv, b1.reshape(1, -1), W2, True)

    parts2 = _scatter_kernel(NCH, NPAD, y2.shape[1])(y2, src3, dst3)

    return _tc_final(parts2, y2, dinv, b2.reshape(1, -1), W_out,
                     b_out.reshape(1, -1))
